# baseline (speedup 1.0000x reference)
import jax
import jax.numpy as jnp
from jax.experimental import pallas as pl
from jax.experimental.pallas import tpu as pltpu

_BIG = 1e8
_NCHAN = 4
_LOG2E = 1.4426950408889634
_LN2 = 0.6931471805599453


def _softmin2(a, b, c):
    m = jnp.minimum(jnp.minimum(a, b), c)
    return m - jnp.log2(jnp.exp2(m - a) + jnp.exp2(m - b) + jnp.exp2(m - c))


def _wavefront_body(bsk_ref, out_ref, dsk_ref, rsk_ref):
    B, N, K2 = bsk_ref.shape
    K = N + N - 1
    for b in range(B):
        for kb in range(K2 // 128):
            blk = bsk_ref[b, :, kb * 128:(kb + 1) * 128]
            dsk_ref[kb * 128:(kb + 1) * 128, b, :] = jnp.swapaxes(blk, 0, 1)
    S = _NCHAN
    c2 = jnp.float32(_LOG2E)
    bigsc = jnp.float32(_BIG * _LOG2E)
    lane = jax.lax.broadcasted_iota(jnp.int32, (B, N), 1)

    def shifted_d(x, s):
        return jnp.where(lane < s, bigsc, jnp.roll(x, s, axis=1))

    d0_0 = dsk_ref[0]
    rsk_ref[0] = d0_0
    c0 = d0_0 * c2
    c1 = shifted_d(c0, 1)
    c2_ = shifted_d(c0, 2)
    c3 = shifted_d(c0, 3)
    p14 = shifted_d(c0, S)
    big_arr = jnp.full((B, N), bigsc, jnp.float32)

    def step(k, carry):
        cur0, cur1, cur2, cur3, q1, q2, q3, p14, p24 = carry
        d0 = dsk_ref[k] * c2
        d1 = shifted_d(d0, 1)
        d2 = shifted_d(d0, 2)
        d3 = shifted_d(d0, 3)
        n0 = d0 + _softmin2(q1, cur1, cur0)
        n1 = d1 + _softmin2(q2, cur2, cur1)
        n2 = d2 + _softmin2(q3, cur3, cur2)
        n3 = d3 + _softmin2(p24, p14, cur3)
        rsk_ref[k] = n0 * jnp.float32(_LN2)
        p14n = jnp.roll(n0, S, axis=1)
        return (n0, n1, n2, n3, cur1, cur2, cur3, p14n, p14)

    jax.lax.fori_loop(
        1, K, step, (c0, c1, c2_, c3, big_arr, big_arr, big_arr, p14, big_arr),
        unroll=16,
    )

    for b in range(B):
        for kb in range(K2 // 128):
            blk = rsk_ref[kb * 128:(kb + 1) * 128, b, :]
            out_ref[b, :, kb * 128:(kb + 1) * 128] = jnp.swapaxes(blk, 0, 1)


def kernel(inputs):
    D = jnp.squeeze(inputs, axis=-1)
    B, N, M = D.shape
    K = N + M - 1
    Dp = jnp.pad(D, ((0, 0), (0, 0), (0, N)), constant_values=_BIG)
    Bsk = Dp.reshape(B, N * (M + N))[:, : N * K].reshape(B, N, K)
    Bsk = jnp.pad(Bsk, ((0, 0), (0, 0), (0, 1)), constant_values=_BIG)

    Ro = pl.pallas_call(
        _wavefront_body,
        out_shape=jax.ShapeDtypeStruct((B, N, M + N), jnp.float32),
        scratch_shapes=[
            pltpu.VMEM((M + N, B, N), jnp.float32),
            pltpu.VMEM((M + N, B, N), jnp.float32),
        ],
    )(Bsk)

    flat = jnp.pad(Ro.reshape(B, N * (M + N)), ((0, 0), (0, N)))
    R = flat.reshape(B, N, M + N + 1)[:, :, :M]
    return jnp.expand_dims(R, axis=-1)


# speedup vs baseline: 1.5039x; 1.5039x over previous
import jax
import jax.numpy as jnp
from jax.experimental import pallas as pl
from jax.experimental.pallas import tpu as pltpu

_BIG = 1e8
_NCHAN = 4
_LOG2E = 1.4426950408889634
_LN2 = 0.6931471805599453


def _softmin2(a, b, c):
    m = jnp.minimum(jnp.minimum(a, b), c)
    return m - jnp.log2(jnp.exp2(m - a) + jnp.exp2(m - b) + jnp.exp2(m - c))


def _sdtw_body(d_ref, out_ref, bsk_ref, dsk_ref, rsk_ref, rskT_ref):
    B, N, M = d_ref.shape
    K = N + M - 1
    K2 = M + N
    S = _NCHAN
    c2 = jnp.float32(_LOG2E)
    bigsc = jnp.float32(_BIG * _LOG2E)
    big512 = jnp.full((B, K2), _BIG, jnp.float32)
    lane = jax.lax.broadcasted_iota(jnp.int32, (B, N), 1)

    for i in range(N):
        bsk_ref[:, i, :] = big512
        bsk_ref[:, i, i:i + M] = d_ref[:, i, :]

    for b in range(B):
        for kb in range(K2 // 128):
            blk = bsk_ref[b, :, kb * 128:(kb + 1) * 128]
            dsk_ref[kb * 128:(kb + 1) * 128, b, :] = jnp.swapaxes(blk, 0, 1)

    def shifted_d(x, s):
        return jnp.where(lane < s, bigsc, jnp.roll(x, s, axis=1))

    d0_0 = dsk_ref[0]
    rsk_ref[0] = d0_0
    c0 = d0_0 * c2
    c1 = shifted_d(c0, 1)
    c2_ = shifted_d(c0, 2)
    c3 = shifted_d(c0, 3)
    p14 = shifted_d(c0, S)
    big_arr = jnp.full((B, N), bigsc, jnp.float32)

    def step(k, carry):
        cur0, cur1, cur2, cur3, q1, q2, q3, p14, p24 = carry
        d0 = dsk_ref[k] * c2
        d1 = shifted_d(d0, 1)
        d2 = shifted_d(d0, 2)
        d3 = shifted_d(d0, 3)
        n0 = d0 + _softmin2(q1, cur1, cur0)
        n1 = d1 + _softmin2(q2, cur2, cur1)
        n2 = d2 + _softmin2(q3, cur3, cur2)
        n3 = d3 + _softmin2(p24, p14, cur3)
        rsk_ref[k] = n0 * jnp.float32(_LN2)
        p14n = jnp.roll(n0, S, axis=1)
        return (n0, n1, n2, n3, cur1, cur2, cur3, p14n, p14)

    jax.lax.fori_loop(
        1, K, step, (c0, c1, c2_, c3, big_arr, big_arr, big_arr, p14, big_arr),
        unroll=16,
    )

    for b in range(B):
        for kb in range(K2 // 128):
            blk = rsk_ref[kb * 128:(kb + 1) * 128, b, :]
            rskT_ref[b, :, kb * 128:(kb + 1) * 128] = jnp.swapaxes(blk, 0, 1)

    for i in range(N):
        out_ref[:, i, :] = rskT_ref[:, i, i:i + M]


def kernel(inputs):
    D = jnp.squeeze(inputs, axis=-1)
    B, N, M = D.shape
    R = pl.pallas_call(
        _sdtw_body,
        out_shape=jax.ShapeDtypeStruct((B, N, M), jnp.float32),
        scratch_shapes=[
            pltpu.VMEM((B, N, M + N), jnp.float32),
            pltpu.VMEM((M + N, B, N), jnp.float32),
            pltpu.VMEM((M + N, B, N), jnp.float32),
            pltpu.VMEM((B, N, M + N), jnp.float32),
        ],
    )(D)
    return jnp.expand_dims(R, axis=-1)
